# baseline (speedup 1.0000x reference)
import functools

import jax
import jax.numpy as jnp
from jax import lax
from jax.experimental import pallas as pl
from jax.experimental.pallas import tpu as pltpu
from jax.experimental.pallas import tpu_sc as plsc

_NL = 16
_CT = 32
_CW = _CT * 128
_NB = 3
_U = 16
_LN2 = 0.6931471805599453
_NEG = float(jnp.finfo(jnp.float32).min)


def _make_sc_kernel(B, V):
    NTF = V // 128
    VR = V - 128 * NTF
    NT = NTF + (1 if VR else 0)
    TS = -(-NT // 8)
    NFULL = -(-TS // _CT) - 1
    assert NFULL == -(-(NTF - 7 * TS) // _CT) - 1 and NFULL % _NB == 0
    NVC = _CW // _NL
    NI = NVC // _U
    mesh = plsc.VectorSubcoreMesh(core_axis_name="c", subcore_axis_name="s")

    @functools.partial(
        pl.kernel,
        mesh=mesh,
        compiler_params=pltpu.CompilerParams(needs_layout_passes=False),
        out_type=[
            jax.ShapeDtypeStruct((B * _NL,), jnp.float32),
            jax.ShapeDtypeStruct((B * _NL,), jnp.int32),
        ],
        name="fixed_categorical_sc",
        scratch_types=[
            pltpu.VMEM((8, _CW), jnp.float32),
            pltpu.VMEM((8, _CW), jnp.float32),
            pltpu.VMEM((8, _CW), jnp.float32),
            pltpu.VMEM((8, VR), jnp.float32),
            pltpu.VMEM((B,), jnp.int32),
            pltpu.VMEM((8, 128), jnp.float32),
            pltpu.VMEM((256,), jnp.float32),
            pltpu.VMEM((128,), jnp.int32),
            pltpu.VMEM((256,), jnp.float32),
            pltpu.VMEM((128,), jnp.int32),
            pltpu.VMEM((_NL,), jnp.float32),
            pltpu.VMEM((_NL,), jnp.int32),
            pltpu.VMEM_SHARED((4096,), jnp.float32),
            pltpu.VMEM_SHARED((2048,), jnp.int32),
            pltpu.SemaphoreType.DMA,
            pltpu.SemaphoreType.DMA,
            pltpu.SemaphoreType.DMA,
            pltpu.SemaphoreType.DMA,
        ],
    )
    def sc_kernel(logits_hbm, actions_hbm, out_lp, out_mode,
                  buf0, buf1, buf2, pbuf, act_v, gat_v, stage_f, stage_i,
                  mslab_f, mslab_i, stf, sti,
                  shared_f, shared_i, sem0, sem1, sem2, semg):
        ci = lax.axis_index("c")
        s = lax.axis_index("s")
        lanes = lax.iota(jnp.int32, _NL)

        gg = s // 8
        j = s % 8
        g_glob = 2 * ci + gg
        rbase = pl.multiple_of(8 * g_glob, 8)
        t0 = j * TS
        t1p = jnp.minimum(TS * (j + 1), NTF)
        te = t1p - _CT
        lo = 128 * (t0 + NFULL * _CT)

        def cslice(tile_idx):
            return pl.ds(pl.multiple_of(128 * tile_idx, 128), _CW)

        row_m = 8 * g_glob + j
        pltpu.sync_copy(actions_hbm, act_v)
        half = jnp.where(jnp.full((_NL,), row_m, jnp.int32) < _NL,
                         act_v[pl.ds(0, _NL)], act_v[pl.ds(_NL, _NL)])
        a = jnp.sum(jnp.where(lanes == (row_m & (_NL - 1)),
                              half.astype(jnp.float32), 0.0)).astype(jnp.int32)
        bt = jnp.minimum(a // 128, NTF - 1)
        gcp = pltpu.async_copy(
            logits_hbm.at[pl.ds(rbase, 8),
                          pl.ds(pl.multiple_of(128 * bt, 128), 128)],
            gat_v, semg)

        pltpu.sync_copy(
            logits_hbm.at[pl.ds(rbase, 8), pl.ds(V - VR, VR)], pbuf)

        bufs = (buf0, buf1, buf2)
        sems = (sem0, sem1, sem2)
        for b in range(_NB):
            pltpu.async_copy(
                logits_hbm.at[pl.ds(rbase, 8), cslice(t0 + b * _CT)],
                bufs[b], sems[b])

        def mrg(m1, x1, m2, x2):
            return (jnp.maximum(m1, m2),
                    jnp.where(m1 > m2, x1,
                              jnp.where(m2 > m1, x2, jnp.minimum(x1, x2))))

        def process(buf, cbase, accs, mask_lo=None):
            new_accs = []
            neg = jnp.full((_NL,), _NEG, jnp.float32)
            zf = jnp.zeros((_NL,), jnp.float32)
            zi = jnp.zeros((_NL,), jnp.int32)
            for r in range(8):
                rm, ri, rs = accs[r]

                def body_a(i, carry, _buf=buf, _r=r):
                    ms = list(carry[:4])
                    xs = list(carry[4:])
                    for u in range(_U):
                        k = u % 4
                        jj = i * _U + u
                        v = _buf[_r, pl.ds(jj * _NL, _NL)]
                        col = cbase + jj * _NL + lanes
                        if mask_lo is not None:
                            v = jnp.where(col >= mask_lo, v, _NEG)
                        upd = v > ms[k]
                        ms[k] = jnp.where(upd, v, ms[k])
                        xs[k] = jnp.where(upd, col, xs[k])
                    return tuple(ms) + tuple(xs)

                ca = lax.fori_loop(0, NI, body_a, (neg,) * 4 + (zi,) * 4)
                m01, x01 = mrg(ca[0], ca[4], ca[1], ca[5])
                m23, x23 = mrg(ca[2], ca[6], ca[3], ca[7])
                cm, cx = mrg(m01, x01, m23, x23)
                new_rm, ri = mrg(rm, ri, cm, cx)
                rs = rs * jnp.exp(rm - new_rm)

                def body_b(i, carry, _buf=buf, _r=r, _m=new_rm):
                    ss = list(carry)
                    for u in range(_U):
                        k = u % 4
                        jj = i * _U + u
                        v = _buf[_r, pl.ds(jj * _NL, _NL)]
                        if mask_lo is not None:
                            col = cbase + jj * _NL + lanes
                            v = jnp.where(col >= mask_lo, v, _NEG)
                        ss[k] = ss[k] + jnp.exp(v - _m)
                    return tuple(ss)

                sb = lax.fori_loop(0, NI, body_b, (zf,) * 4)
                rs = rs + ((sb[0] + sb[1]) + (sb[2] + sb[3]))
                new_accs.append((new_rm, ri, rs))
            return new_accs

        def outer(g, flat):
            accs = [(flat[3 * r], flat[3 * r + 1], flat[3 * r + 2])
                    for r in range(8)]
            for b in range(_NB):
                k = _NB * g + b
                tk = t0 + _CT * k
                pltpu.make_async_copy(
                    logits_hbm.at[pl.ds(rbase, 8), cslice(tk)],
                    bufs[b], sems[b]).wait()
                accs = process(bufs[b], 128 * tk, accs)
                if b == 0:
                    tnext = jnp.where(k + _NB < NFULL, t0 + _CT * (k + _NB),
                                      te)
                    pltpu.async_copy(
                        logits_hbm.at[pl.ds(rbase, 8), cslice(tnext)],
                        bufs[0], sems[0])
                else:
                    @pl.when(g < NFULL // _NB - 1)
                    def _():
                        pltpu.async_copy(
                            logits_hbm.at[pl.ds(rbase, 8),
                                          cslice(t0 + _CT * (k + _NB))],
                            bufs[b], sems[b])
            return tuple(x for acc in accs for x in acc)

        init = []
        for _ in range(8):
            init += [jnp.full((_NL,), _NEG, jnp.float32),
                     jnp.zeros((_NL,), jnp.int32),
                     jnp.zeros((_NL,), jnp.float32)]
        flat = lax.fori_loop(0, NFULL // _NB, outer, tuple(init))
        accs = [(flat[3 * r], flat[3 * r + 1], flat[3 * r + 2])
                for r in range(8)]

        pltpu.make_async_copy(
            logits_hbm.at[pl.ds(rbase, 8), cslice(te)],
            bufs[0], sems[0]).wait()
        accs = process(bufs[0], 128 * te, accs, mask_lo=lo)

        is_j7 = j == 7
        new_accs = []
        for r in range(8):
            rm, ri, rs = accs[r]
            for x in range(VR // _NL):
                v = jnp.where(is_j7, pbuf[r, pl.ds(x * _NL, _NL)], _NEG)
                col = (V - VR) + x * _NL + lanes
                upd = v > rm
                rm2 = jnp.where(upd, v, rm)
                ri = jnp.where(upd, col, ri)
                rs = rs * jnp.exp(rm - rm2) + jnp.exp(v - rm2)
                rm = rm2
            new_accs.append((rm, ri, rs))

        for r in range(8):
            rm, ri, rs = new_accs[r]
            stage_f[pl.ds(r * 32, _NL)] = rm
            stage_f[pl.ds(r * 32 + _NL, _NL)] = rs
            stage_i[pl.ds(r * _NL, _NL)] = ri
        pltpu.sync_copy(stage_f, shared_f.at[pl.ds(j * 512 + gg * 256, 256)])
        pltpu.sync_copy(stage_i, shared_i.at[pl.ds(j * 256 + gg * 128, 128)])
        plsc.subcore_barrier()

        for jj in range(8):
            pltpu.sync_copy(shared_f.at[pl.ds(jj * 512 + s * 32, 32)],
                            mslab_f.at[pl.ds(jj * 32, 32)])
            pltpu.sync_copy(shared_i.at[pl.ds(jj * 256 + s * _NL, _NL)],
                            mslab_i.at[pl.ds(jj * _NL, _NL)])

        m_vec = jnp.full((_NL,), _NEG, jnp.float32)
        for jj in range(8):
            m_vec = jnp.maximum(m_vec, mslab_f[pl.ds(jj * 32, _NL)])
        s_vec = jnp.zeros((_NL,), jnp.float32)
        i_vec = jnp.full((_NL,), V, jnp.int32)
        for jj in range(8):
            rm_j = mslab_f[pl.ds(jj * 32, _NL)]
            rs_j = mslab_f[pl.ds(jj * 32 + _NL, _NL)]
            ri_j = mslab_i[pl.ds(jj * _NL, _NL)]
            s_vec = s_vec + rs_j * jnp.exp(rm_j - m_vec)
            i_vec = jnp.minimum(i_vec, jnp.where(rm_j == m_vec, ri_j, V))

        m_row = jnp.max(m_vec)
        s_row = jnp.sum(s_vec * jnp.exp(m_vec - m_row))
        mode = jnp.min(jnp.where(m_vec == m_row, i_vec.astype(jnp.float32),
                                 float(V))).astype(jnp.int32)

        gcp.wait()
        r8 = j
        ac = a - 128 * bt
        w0 = pl.multiple_of((ac // _NL) * _NL, _NL)
        lane0 = ac - w0
        vm = jnp.zeros((_NL,), jnp.float32)
        for rr in range(8):
            vm = vm + jnp.where(jnp.full((_NL,), r8, jnp.int32) == rr,
                                gat_v[rr, pl.ds(w0, _NL)], 0.0)
        val_m = jnp.sum(jnp.where(lanes == lane0, vm, 0.0))
        a2 = a - (V - VR)
        a2c = jnp.maximum(a2, 0)
        w1 = pl.multiple_of((a2c // _NL) * _NL, _NL)
        lane1 = a2c - w1
        vp = jnp.zeros((_NL,), jnp.float32)
        for rr in range(8):
            vp = vp + jnp.where(jnp.full((_NL,), r8, jnp.int32) == rr,
                                pbuf[rr, pl.ds(w1, _NL)], 0.0)
        val_p = jnp.sum(jnp.where(lanes == lane1, vp, 0.0))
        val = jnp.where(a2 >= 0, val_p, val_m)

        sv = jnp.full((_NL,), s_row, jnp.float32)
        bits = plsc.bitcast(sv, jnp.int32)
        e = lax.shift_right_logical(bits, 23) - 127
        mant = plsc.bitcast((bits & 0x7FFFFF) | (127 << 23), jnp.float32)
        y = e.astype(jnp.float32) * _LN2 + (mant - 1.0) * 0.7
        for _ in range(4):
            y = y - 1.0 + sv * jnp.exp(-y)

        stf[...] = (val - m_row) - y
        sti[...] = jnp.full((_NL,), mode, jnp.int32)
        pltpu.sync_copy(stf, out_lp.at[pl.ds(row_m * _NL, _NL)])
        pltpu.sync_copy(sti, out_mode.at[pl.ds(row_m * _NL, _NL)])

    return sc_kernel


@jax.jit
def kernel(logits, actions):
    B, V = logits.shape
    sck = _make_sc_kernel(B, V)
    out_lp, out_mode = sck(logits, actions.reshape(B))
    return (out_lp.reshape(B, _NL)[:, :1], out_mode.reshape(B, _NL)[:, :1])


# speedup vs baseline: 1.1453x; 1.1453x over previous
import functools

import jax
import jax.numpy as jnp
from jax import lax
from jax.experimental import pallas as pl
from jax.experimental.pallas import tpu as pltpu
from jax.experimental.pallas import tpu_sc as plsc

_NL = 16
_CT = 48
_CW = _CT * 128
_NB = 2
_UV = 8
_LN2 = 0.6931471805599453
_NEG = float(jnp.finfo(jnp.float32).min)


def _make_sc_kernel(B, V):
    NTF = V // 128
    VR = V - 128 * NTF
    NT = NTF + (1 if VR else 0)
    TS = -(-NT // 8)
    NFULL = -(-TS // _CT) - 1
    assert NFULL == -(-(NTF - 7 * TS) // _CT) - 1 and NFULL % _NB == 0
    NVC = _CW // _NL
    NI = NVC // _UV
    mesh = plsc.VectorSubcoreMesh(core_axis_name="c", subcore_axis_name="s")

    @functools.partial(
        pl.kernel,
        mesh=mesh,
        compiler_params=pltpu.CompilerParams(needs_layout_passes=False),
        out_type=[
            jax.ShapeDtypeStruct((B * _NL,), jnp.float32),
            jax.ShapeDtypeStruct((B * _NL,), jnp.int32),
        ],
        name="fixed_categorical_sc",
        scratch_types=[pltpu.VMEM((8, _CW), jnp.float32)] * _NB + [
            pltpu.VMEM((8, VR), jnp.float32),
            pltpu.VMEM((B,), jnp.int32),
            pltpu.VMEM((8, 128), jnp.float32),
            pltpu.VMEM((256,), jnp.float32),
            pltpu.VMEM((128,), jnp.int32),
            pltpu.VMEM((256,), jnp.float32),
            pltpu.VMEM((128,), jnp.int32),
            pltpu.VMEM((_NL,), jnp.float32),
            pltpu.VMEM((_NL,), jnp.int32),
            pltpu.VMEM_SHARED((4096,), jnp.float32),
            pltpu.VMEM_SHARED((2048,), jnp.int32),
        ] + [pltpu.SemaphoreType.DMA] * (_NB + 1),
    )
    def sc_kernel(logits_hbm, actions_hbm, out_lp, out_mode,
                  *refs):
        bufs = refs[:_NB]
        (pbuf, act_v, gat_v, stage_f, stage_i,
         mslab_f, mslab_i, stf, sti, shared_f, shared_i) = refs[_NB:_NB + 11]
        sems = refs[_NB + 11:-1]
        semg = refs[-1]
        ci = lax.axis_index("c")
        s = lax.axis_index("s")
        lanes = lax.iota(jnp.int32, _NL)

        gg = s // 8
        j = s % 8
        g_glob = 2 * ci + gg
        rbase = pl.multiple_of(8 * g_glob, 8)
        t0 = j * TS
        t1p = jnp.minimum(TS * (j + 1), NTF)
        te = t1p - _CT
        lo = 128 * (t0 + NFULL * _CT)

        def cslice(tile_idx):
            return pl.ds(pl.multiple_of(128 * tile_idx, 128), _CW)

        row_m = 8 * g_glob + j
        pltpu.sync_copy(actions_hbm, act_v)
        half = jnp.where(jnp.full((_NL,), row_m, jnp.int32) < _NL,
                         act_v[pl.ds(0, _NL)], act_v[pl.ds(_NL, _NL)])
        a = jnp.sum(jnp.where(lanes == (row_m & (_NL - 1)),
                              half.astype(jnp.float32), 0.0)).astype(jnp.int32)
        bt = jnp.minimum(a // 128, NTF - 1)
        gcp = pltpu.async_copy(
            logits_hbm.at[pl.ds(rbase, 8),
                          pl.ds(pl.multiple_of(128 * bt, 128), 128)],
            gat_v, semg)

        pltpu.sync_copy(
            logits_hbm.at[pl.ds(rbase, 8), pl.ds(V - VR, VR)], pbuf)

        for b in range(_NB):
            pltpu.async_copy(
                logits_hbm.at[pl.ds(rbase, 8), cslice(t0 + b * _CT)],
                bufs[b], sems[b])

        def mrg(m1, x1, m2, x2):
            return (jnp.maximum(m1, m2),
                    jnp.where(m1 > m2, x1,
                              jnp.where(m2 > m1, x2, jnp.minimum(x1, x2))))

        def process(buf, cbase, accs, mask_lo=None):
            new_accs = [None] * 8
            neg = jnp.full((_NL,), _NEG, jnp.float32)
            zf = jnp.zeros((_NL,), jnp.float32)
            zi = jnp.zeros((_NL,), jnp.int32)
            for r0 in range(0, 8, 2):

                def body_a(i, carry, _buf=buf, _r0=r0):
                    ms = [list(carry[0:4]), list(carry[4:8])]
                    xs = [list(carry[8:12]), list(carry[12:16])]
                    for u in range(_UV):
                        k = u % 4
                        jj = i * _UV + u
                        col = cbase + jj * _NL + lanes
                        for p in range(2):
                            v = _buf[_r0 + p, pl.ds(jj * _NL, _NL)]
                            if mask_lo is not None:
                                v = jnp.where(col >= mask_lo, v, _NEG)
                            upd = v > ms[p][k]
                            ms[p][k] = jnp.where(upd, v, ms[p][k])
                            xs[p][k] = jnp.where(upd, col, xs[p][k])
                    return (tuple(ms[0]) + tuple(ms[1])
                            + tuple(xs[0]) + tuple(xs[1]))

                ca = lax.fori_loop(0, NI, body_a, (neg,) * 8 + (zi,) * 8)
                nrm = [None, None]
                for p in range(2):
                    mv = ca[4 * p:4 * p + 4]
                    xv = ca[8 + 4 * p:12 + 4 * p]
                    m01, x01 = mrg(mv[0], xv[0], mv[1], xv[1])
                    m23, x23 = mrg(mv[2], xv[2], mv[3], xv[3])
                    cm, cx = mrg(m01, x01, m23, x23)
                    rm, ri, rs = accs[r0 + p]
                    new_rm, ri = mrg(rm, ri, cm, cx)
                    rs = rs * jnp.exp(rm - new_rm)
                    nrm[p] = new_rm
                    new_accs[r0 + p] = (new_rm, ri, rs)

                def body_b(i, carry, _buf=buf, _r0=r0, _m=tuple(nrm)):
                    ss = [list(carry[0:4]), list(carry[4:8])]
                    for u in range(_UV):
                        k = u % 4
                        jj = i * _UV + u
                        for p in range(2):
                            v = _buf[_r0 + p, pl.ds(jj * _NL, _NL)]
                            if mask_lo is not None:
                                col = cbase + jj * _NL + lanes
                                v = jnp.where(col >= mask_lo, v, _NEG)
                            ss[p][k] = ss[p][k] + jnp.exp(v - _m[p])
                    return tuple(ss[0]) + tuple(ss[1])

                sb = lax.fori_loop(0, NI, body_b, (zf,) * 8)
                for p in range(2):
                    new_rm, ri, rs = new_accs[r0 + p]
                    sp = sb[4 * p:4 * p + 4]
                    rs = rs + ((sp[0] + sp[1]) + (sp[2] + sp[3]))
                    new_accs[r0 + p] = (new_rm, ri, rs)
            return new_accs

        def outer(g, flat):
            accs = [(flat[3 * r], flat[3 * r + 1], flat[3 * r + 2])
                    for r in range(8)]
            for b in range(_NB):
                k = _NB * g + b
                tk = t0 + _CT * k
                pltpu.make_async_copy(
                    logits_hbm.at[pl.ds(rbase, 8), cslice(tk)],
                    bufs[b], sems[b]).wait()
                accs = process(bufs[b], 128 * tk, accs)
                if b == 0:
                    tnext = jnp.where(k + _NB < NFULL, t0 + _CT * (k + _NB),
                                      te)
                    pltpu.async_copy(
                        logits_hbm.at[pl.ds(rbase, 8), cslice(tnext)],
                        bufs[0], sems[0])
                else:
                    @pl.when(g < NFULL // _NB - 1)
                    def _():
                        pltpu.async_copy(
                            logits_hbm.at[pl.ds(rbase, 8),
                                          cslice(t0 + _CT * (k + _NB))],
                            bufs[b], sems[b])
            return tuple(x for acc in accs for x in acc)

        init = []
        for _ in range(8):
            init += [jnp.full((_NL,), _NEG, jnp.float32),
                     jnp.zeros((_NL,), jnp.int32),
                     jnp.zeros((_NL,), jnp.float32)]
        flat = lax.fori_loop(0, NFULL // _NB, outer, tuple(init))
        accs = [(flat[3 * r], flat[3 * r + 1], flat[3 * r + 2])
                for r in range(8)]

        pltpu.make_async_copy(
            logits_hbm.at[pl.ds(rbase, 8), cslice(te)],
            bufs[0], sems[0]).wait()
        accs = process(bufs[0], 128 * te, accs, mask_lo=lo)

        is_j7 = j == 7
        new_accs = []
        for r in range(8):
            rm, ri, rs = accs[r]
            for x in range(VR // _NL):
                v = jnp.where(is_j7, pbuf[r, pl.ds(x * _NL, _NL)], _NEG)
                col = (V - VR) + x * _NL + lanes
                upd = v > rm
                rm2 = jnp.where(upd, v, rm)
                ri = jnp.where(upd, col, ri)
                rs = rs * jnp.exp(rm - rm2) + jnp.exp(v - rm2)
                rm = rm2
            new_accs.append((rm, ri, rs))

        for r in range(8):
            rm, ri, rs = new_accs[r]
            stage_f[pl.ds(r * 32, _NL)] = rm
            stage_f[pl.ds(r * 32 + _NL, _NL)] = rs
            stage_i[pl.ds(r * _NL, _NL)] = ri
        pltpu.sync_copy(stage_f, shared_f.at[pl.ds(j * 512 + gg * 256, 256)])
        pltpu.sync_copy(stage_i, shared_i.at[pl.ds(j * 256 + gg * 128, 128)])
        plsc.subcore_barrier()

        for jj in range(8):
            pltpu.sync_copy(shared_f.at[pl.ds(jj * 512 + s * 32, 32)],
                            mslab_f.at[pl.ds(jj * 32, 32)])
            pltpu.sync_copy(shared_i.at[pl.ds(jj * 256 + s * _NL, _NL)],
                            mslab_i.at[pl.ds(jj * _NL, _NL)])

        m_vec = jnp.full((_NL,), _NEG, jnp.float32)
        for jj in range(8):
            m_vec = jnp.maximum(m_vec, mslab_f[pl.ds(jj * 32, _NL)])
        s_vec = jnp.zeros((_NL,), jnp.float32)
        i_vec = jnp.full((_NL,), V, jnp.int32)
        for jj in range(8):
            rm_j = mslab_f[pl.ds(jj * 32, _NL)]
            rs_j = mslab_f[pl.ds(jj * 32 + _NL, _NL)]
            ri_j = mslab_i[pl.ds(jj * _NL, _NL)]
            s_vec = s_vec + rs_j * jnp.exp(rm_j - m_vec)
            i_vec = jnp.minimum(i_vec, jnp.where(rm_j == m_vec, ri_j, V))

        m_row = jnp.max(m_vec)
        s_row = jnp.sum(s_vec * jnp.exp(m_vec - m_row))
        mode = jnp.min(jnp.where(m_vec == m_row, i_vec.astype(jnp.float32),
                                 float(V))).astype(jnp.int32)

        gcp.wait()
        r8 = j
        ac = a - 128 * bt
        w0 = pl.multiple_of((ac // _NL) * _NL, _NL)
        lane0 = ac - w0
        vm = jnp.zeros((_NL,), jnp.float32)
        for rr in range(8):
            vm = vm + jnp.where(jnp.full((_NL,), r8, jnp.int32) == rr,
                                gat_v[rr, pl.ds(w0, _NL)], 0.0)
        val_m = jnp.sum(jnp.where(lanes == lane0, vm, 0.0))
        a2 = a - (V - VR)
        a2c = jnp.maximum(a2, 0)
        w1 = pl.multiple_of((a2c // _NL) * _NL, _NL)
        lane1 = a2c - w1
        vp = jnp.zeros((_NL,), jnp.float32)
        for rr in range(8):
            vp = vp + jnp.where(jnp.full((_NL,), r8, jnp.int32) == rr,
                                pbuf[rr, pl.ds(w1, _NL)], 0.0)
        val_p = jnp.sum(jnp.where(lanes == lane1, vp, 0.0))
        val = jnp.where(a2 >= 0, val_p, val_m)

        sv = jnp.full((_NL,), s_row, jnp.float32)
        bits = plsc.bitcast(sv, jnp.int32)
        e = lax.shift_right_logical(bits, 23) - 127
        mant = plsc.bitcast((bits & 0x7FFFFF) | (127 << 23), jnp.float32)
        y = e.astype(jnp.float32) * _LN2 + (mant - 1.0) * 0.7
        for _ in range(4):
            y = y - 1.0 + sv * jnp.exp(-y)

        stf[...] = (val - m_row) - y
        sti[...] = jnp.full((_NL,), mode, jnp.int32)
        pltpu.sync_copy(stf, out_lp.at[pl.ds(row_m * _NL, _NL)])
        pltpu.sync_copy(sti, out_mode.at[pl.ds(row_m * _NL, _NL)])

    return sc_kernel


@jax.jit
def kernel(logits, actions):
    B, V = logits.shape
    sck = _make_sc_kernel(B, V)
    out_lp, out_mode = sck(logits, actions.reshape(B))
    return (out_lp.reshape(B, _NL)[:, :1], out_mode.reshape(B, _NL)[:, :1])
